# baseline (speedup 1.0000x reference)
import jax
import jax.numpy as jnp
from jax import lax
from jax.experimental import pallas as pl
from jax.experimental.pallas import tpu as pltpu
from jax.experimental.pallas import tpu_sc as plsc

NC, NS, L = 2, 16, 16
NW = NC * NS
B, S, M, D = 4096, 50, 5, 64
DP = 128
BATCHES_W = B // NW
CHUNK = S * M
NCHUNK = BATCHES_W
QUAD = 4 * CHUNK
QPAD = 1008
NBUF = 2
PIECE = 128
NROWS = 256
N = B * S * M


def _body(idx_hbm, table_hbm, out3_hbm, idx_v, idx_h, rows_v, out_v,
          sem0, sem1):
  sems = (sem0, sem1)
  wid = lax.axis_index("s") * NC + lax.axis_index("c")
  wb0 = wid * BATCHES_W
  wi0 = wid * BATCHES_W * CHUNK
  iota = lax.iota(jnp.int32, L)
  iota5 = iota * 5
  tail_mask = iota < (S - 3 * L)

  def load_quad(q, qs):
    pltpu.sync_copy(idx_hbm.at[pl.ds(wi0 + q * QUAD, QUAD)],
                    idx_v.at[qs, pl.ds(0, QUAD)])

    def halve(i, carry):
      sl = pl.ds(i * L, L)
      idx_h[qs, sl] = lax.shift_right_logical(idx_v[qs, sl], 1)
      return carry

    lax.fori_loop(0, QPAD // L, halve, 0, unroll=4)

  def fire(b, k, qs):
    for j in range(2):
      soff = 248 * k + j * PIECE
      pltpu.async_copy(
          table_hbm.at[idx_h.at[qs, pl.ds(soff, PIECE)]],
          rows_v.at[b, pl.ds(j * PIECE, PIECE)],
          sems[b],
      )

  def drain(b):
    pltpu.make_async_copy(
        table_hbm.at[pl.ds(0, NROWS)], rows_v.at[b], sems[b]
    ).wait()

  def reduce_store(b, c, k, qs):
    q0 = 250 * k
    rows = rows_v.at[b]
    for gb in range(4):
      mask = None if gb < 3 else tail_mask
      rvecs, cbases = [], []
      for m in range(M):
        gidx = plsc.load_gather(
            idx_v.at[qs], [iota5 + (q0 + 80 * gb + m)], mask=mask)
        rvecs.append(iota5 + (2 * k + 80 * gb + m))
        cbases.append((gidx & 1) * 64)
      gvec = iota + 16 * gb

      def dloop(dd, dvec):
        acc = plsc.load_gather(rows, [rvecs[0], cbases[0] + dvec], mask=mask)
        for m in range(1, M):
          acc = acc + plsc.load_gather(
              rows, [rvecs[m], cbases[m] + dvec], mask=mask)
        plsc.store_scatter(out_v, [gvec, dvec], acc * (1.0 / M), mask=mask)
        return dvec + 1

      lax.fori_loop(0, D, dloop, jnp.zeros((L,), jnp.int32), unroll=2)
    pltpu.sync_copy(out_v, out3_hbm.at[wb0 + c])

  load_quad(0, 0)
  fire(0, 0, 0)
  fire(1, 1, 0)

  def step(s, carry):
    qs_cur = s & 1
    qs_next = (s + 1) & 1
    for k in range(4):
      b = k & 1
      c = 4 * s + k
      drain(b)
      reduce_store(b, c, k, qs_cur)
      cn = c + NBUF
      kn = (k + NBUF) & 3
      qn = qs_next if k >= 2 else qs_cur

      @pl.when(cn < NCHUNK)
      def _():
        if k == 2:
          load_quad(s + 1, qs_next)
        fire(b, kn, qn)

    return carry

  lax.fori_loop(0, NCHUNK // 4, step, 0)


_sc_call = pl.kernel(
    _body,
    out_type=jax.ShapeDtypeStruct((B, S, D), jnp.float32),
    mesh=plsc.VectorSubcoreMesh(
        core_axis_name="c", subcore_axis_name="s", num_cores=NC,
        num_subcores=NS),
    scratch_types=[
        pltpu.VMEM((2, QPAD), jnp.int32),
        pltpu.VMEM((2, QPAD), jnp.int32),
        pltpu.VMEM((NBUF, NROWS, DP), jnp.float32),
        pltpu.VMEM((S, D), jnp.float32),
        pltpu.SemaphoreType.DMA,
        pltpu.SemaphoreType.DMA,
    ],
    compiler_params=pltpu.CompilerParams(
        use_tc_tiling_on_sc=False, needs_layout_passes=False),
)


def kernel(meta_indices, table):
  table2 = table.reshape(table.shape[0] // 2, DP)
  return _sc_call(meta_indices.astype(jnp.int32).reshape(N), table2)


# speedup vs baseline: 2.2771x; 2.2771x over previous
import jax
import jax.numpy as jnp
from jax import lax
from jax.experimental import pallas as pl
from jax.experimental.pallas import tpu as pltpu
from jax.experimental.pallas import tpu_sc as plsc

NC, NS, L = 2, 16, 16
NW = NC * NS
B, S, M, D = 4096, 50, 5, 64
DP = 128
BATCHES_W = B // NW
CHUNK = S * M
NCHUNK = BATCHES_W
QUAD = 4 * CHUNK
NBUF = 2
PIECE = 128
NROWS = 256
N = B * S * M


def _body(idx_hbm, table_hbm, out3_hbm, idx_v, rows_v, out_v, sem0, sem1):
  sems = (sem0, sem1)
  wid = lax.axis_index("s") * NC + lax.axis_index("c")
  wb0 = wid * BATCHES_W
  wi0 = wid * BATCHES_W * CHUNK

  def load_quad(q, qs):
    pltpu.sync_copy(idx_hbm.at[pl.ds(wi0 + q * QUAD, QUAD)], idx_v.at[qs])

  def fire(b, k, qs):
    for j in range(2):
      soff = 248 * k + j * PIECE
      pltpu.async_copy(
          table_hbm.at[idx_v.at[qs, pl.ds(soff, PIECE)]],
          rows_v.at[b, pl.ds(j * PIECE, PIECE)],
          sems[b],
      )

  def drain(b):
    pltpu.make_async_copy(
        table_hbm.at[pl.ds(0, NROWS)], rows_v.at[b], sems[b]
    ).wait()

  def reduce_store(b, c, k):
    ro = 2 * k

    def grp(ss, carry):
      r = ro + ss * M
      for d in range(D // L):
        sl = pl.ds(d * L, L)
        acc = rows_v[b, r, sl]
        for m in range(1, M):
          acc = acc + rows_v[b, r + m, sl]
        out_v[ss, sl] = acc * (1.0 / M)
      return carry

    lax.fori_loop(0, S, grp, 0, unroll=2)
    pltpu.sync_copy(out_v, out3_hbm.at[wb0 + c])

  load_quad(0, 0)
  fire(0, 0, 0)
  fire(1, 1, 0)

  def step(s, carry):
    qs_cur = s & 1
    qs_next = (s + 1) & 1
    for k in range(4):
      b = k & 1
      c = 4 * s + k
      drain(b)
      reduce_store(b, c, k)
      cn = c + NBUF
      kn = (k + NBUF) & 3
      qn = qs_next if k >= 2 else qs_cur

      @pl.when(cn < NCHUNK)
      def _():
        if k == 2:
          load_quad(s + 1, qs_next)
        fire(b, kn, qn)

    return carry

  lax.fori_loop(0, NCHUNK // 4, step, 0)


_sc_call = pl.kernel(
    _body,
    out_type=jax.ShapeDtypeStruct((B, S, D), jnp.float32),
    mesh=plsc.VectorSubcoreMesh(
        core_axis_name="c", subcore_axis_name="s", num_cores=NC,
        num_subcores=NS),
    scratch_types=[
        pltpu.VMEM((2, QUAD), jnp.int32),
        pltpu.VMEM((NBUF, NROWS, DP), jnp.float32),
        pltpu.VMEM((S, D), jnp.float32),
        pltpu.SemaphoreType.DMA,
        pltpu.SemaphoreType.DMA,
    ],
    compiler_params=pltpu.CompilerParams(use_tc_tiling_on_sc=False),
)


def kernel(meta_indices, table):
  table_p = jnp.pad(table, ((0, 0), (0, DP - D)))
  return _sc_call(meta_indices.astype(jnp.int32).reshape(N), table_p)
